# baseline (speedup 1.0000x reference)
import functools

import jax
import jax.numpy as jnp
from jax import lax
from jax.experimental import pallas as pl
from jax.experimental.pallas import tpu as pltpu
from jax.experimental.pallas import tpu_sc as plsc

RX = 4
RY = 4
NC = 2
NS = 16
L = 16
SLOTS = 16
SW = 24
WIN = SW + 10
RPS = 2
GRP = RPS * SW // L
TAPS = [(j - RY // 2, i - RX // 2) for j in range(RY) for i in range(RX)]
NT = len(TAPS)


def _sc_call(N, C, H, W):
    CH = C // 2
    CPI = CH + 1
    n_steps = H // RPS

    mesh = plsc.VectorSubcoreMesh(
        core_axis_name="c", subcore_axis_name="s", num_cores=NC,
        num_subcores=NS)

    @functools.partial(
        pl.kernel,
        mesh=mesh,
        compiler_params=pltpu.CompilerParams(
            needs_layout_passes=False, use_tc_tiling_on_sc=False),
        out_type=jax.ShapeDtypeStruct(
            (N, NS, H // RPS, NT, RPS, SW), jnp.float32),
        scratch_types=[
            pltpu.VMEM((SLOTS * WIN, CPI), jnp.int32),
            pltpu.VMEM((2, C, RPS * SW), jnp.float32),
            pltpu.VMEM((2, 2, RPS, SW), jnp.int32),
            pltpu.VMEM((2, NT, RPS, SW), jnp.float32),
        ] + [pltpu.SemaphoreType.DMA] * 8,
    )
    def call(f1_hbm, f2_hbm, ofs_hbm, out_hbm, win, f1b, ofsb, outb,
             rs0, rs1, fs0, fs1, os0, os1, ws0, ws1):
        n = lax.axis_index("c")
        tec = lax.axis_index("s")
        x0 = tec * SW
        xs = jnp.clip(x0 - 2, 0, W - WIN)
        img = n * H * W
        lane = lax.iota(jnp.int32, L)
        zero_i = lane * 0
        one_i = zero_i + 1
        r1 = (lane >= (2 * L - SW)).astype(jnp.int32)
        lane_r = [zero_i, r1, one_i]
        lane_x = [lane, lane + L - SW * r1, lane + 2 * L - SW]
        tap_id = [zero_i + t for t in range(NT)]
        comp0 = zero_i
        comp1 = one_i

        def ring_fire(row, sem):
            r = jnp.minimum(row, H - 1)
            return pltpu.async_copy(
                f2_hbm.at[pl.ds(img + r * W + xs, WIN), :],
                win.at[pl.ds((r & (SLOTS - 1)) * WIN, WIN), :], sem)

        def f1_fire(s, buf, sem):
            ss = jnp.minimum(s, H // RPS - 1)
            pltpu.async_copy(f1_hbm.at[n, tec, ss], f1b.at[buf], sem)

        def ofs_fire(s, buf, sem):
            ss = jnp.minimum(s, H // RPS - 1)
            pltpu.async_copy(ofs_hbm.at[n, tec, ss], ofsb.at[buf], sem)

        def out_fire(s, buf, sem):
            return pltpu.async_copy(
                outb.at[buf], out_hbm.at[n, tec, s], sem)

        def ring_wait(sem):
            pltpu.make_async_copy(
                f2_hbm.at[pl.ds(0, WIN), :],
                win.at[pl.ds(0, WIN), :], sem).wait()

        def f1_wait(sem):
            pltpu.make_async_copy(
                f1_hbm.at[0, 0, 0], f1b.at[0], sem).wait()

        def ofs_wait(sem):
            pltpu.make_async_copy(
                ofs_hbm.at[0, 0, 0], ofsb.at[0], sem).wait()

        def out_wait(sem):
            pltpu.make_async_copy(
                outb.at[0], out_hbm.at[0, 0, 0], sem).wait()

        for k in range(SLOTS - 4):
            pltpu.sync_copy(
                f2_hbm.at[pl.ds(img + k * W + xs, WIN), :],
                win.at[pl.ds(k * WIN, WIN), :])
        pltpu.sync_copy(f1_hbm.at[n, tec, 0], f1b.at[0])
        pltpu.sync_copy(ofs_hbm.at[n, tec, 0], ofsb.at[0])

        def compute(y, p):
            for g in range(GRP):
                ofx = plsc.load_gather(
                    ofsb.at[p], [comp0, lane_r[g], lane_x[g]])
                ofy = plsc.load_gather(
                    ofsb.at[p], [comp1, lane_r[g], lane_x[g]])
                gx0 = x0 + lane_x[g] + ofx
                gy0 = y + lane_r[g] + ofy
                winrow = []
                for (dy, dx) in TAPS:
                    sxl = jnp.clip((gx0 + dx) - xs, 0, WIN - 1)
                    slot = (gy0 + dy) & (SLOTS - 1)
                    winrow.append(slot * WIN + sxl)
                xg = g * L

                def c_body(c2, accs):
                    cs = comp0 + c2
                    f1e = f1b[p, 2 * c2, pl.ds(xg, L)]
                    f1o = f1b[p, 2 * c2 + 1, pl.ds(xg, L)]
                    new = []
                    for t in range(NT):
                        w = plsc.load_gather(win, [winrow[t], cs])
                        lo, hi = plsc.unpack(
                            plsc.bitcast(w, jnp.bfloat16),
                            format=plsc.PackFormat.INTERLEAVED)
                        new.append(accs[t] + (f1e * lo + f1o * hi))
                    return tuple(new)

                accs = lax.fori_loop(
                    0, CH, c_body,
                    tuple(jnp.zeros((L,), jnp.float32) for _ in range(NT)))
                zero = jnp.zeros((L,), jnp.float32)
                for t, (dy, dx) in enumerate(TAPS):
                    gx = plsc.bitcast(gx0 + dx, jnp.uint32)
                    gy = plsc.bitcast(gy0 + dy, jnp.uint32)
                    m = (gx < W) & (gy < H)
                    plsc.store_scatter(
                        outb.at[p], [tap_id[t], lane_r[g], lane_x[g]],
                        jnp.where(m, accs[t], zero))

        def step(t, _):
            for p in range(2):
                s = 2 * t + p
                y = RPS * s
                not_first = t >= 1

                if p == 0:
                    @pl.when(not_first)
                    def _():
                        f1_wait(fs0)
                        ofs_wait(os0)
                else:
                    f1_wait(fs1)
                    ofs_wait(os1)

                @pl.when(not_first)
                def _():
                    ring_wait(rs0 if p == 0 else rs1)
                    ring_wait(rs0 if p == 0 else rs1)
                    out_wait(ws0 if p == 0 else ws1)

                ring_fire(y + 12, rs0 if p == 0 else rs1)
                ring_fire(y + 13, rs0 if p == 0 else rs1)
                f1_fire(s + 1, 1 - p, fs1 if p == 0 else fs0)
                ofs_fire(s + 1, 1 - p, os1 if p == 0 else os0)

                compute(y, p)
                out_fire(s, p, ws0 if p == 0 else ws1)
            return 0

        lax.fori_loop(0, n_steps // 2, step, 0)

        ring_wait(rs0)
        ring_wait(rs0)
        ring_wait(rs1)
        ring_wait(rs1)
        f1_wait(fs0)
        ofs_wait(os0)
        out_wait(ws0)
        out_wait(ws1)

    return call


def kernel(f1, f2, ofs):
    N, C, H, W = f1.shape
    H2 = H // RPS
    assert N == NC and W == NS * SW and H % (2 * RPS) == 0 and C % 2 == 0
    f2p = lax.bitcast_convert_type(
        jnp.transpose(f2, (0, 2, 3, 1)).astype(jnp.bfloat16).reshape(
            N * H * W, C // 2, 2),
        jnp.int32)
    f2p = jnp.concatenate(
        [f2p, jnp.zeros((N * H * W, 1), jnp.int32)], axis=1)
    f1q = f1.reshape(N, C, H2, RPS, NS, SW).transpose(
        0, 4, 2, 1, 3, 5).reshape(N, NS, H2, C, RPS * SW)
    ofs_q = ofs.reshape(N, 2, H2, RPS, NS, SW).transpose(0, 4, 2, 1, 3, 5)
    out_t = _sc_call(N, C, H, W)(f1q, f2p, ofs_q)
    return out_t.transpose(0, 3, 2, 4, 1, 5).reshape(N, NT, H, W)


# speedup vs baseline: 1.0816x; 1.0816x over previous
import functools

import jax
import jax.numpy as jnp
from jax import lax
from jax.experimental import pallas as pl
from jax.experimental.pallas import tpu as pltpu
from jax.experimental.pallas import tpu_sc as plsc

RX = 4
RY = 4
NC = 2
NS = 16
L = 16
SLOTS = 16
SW = 24
WIN = SW + 10
RPS = 2
GRP = RPS * SW // L
TAPS = [(j - RY // 2, i - RX // 2) for j in range(RY) for i in range(RX)]
NT = len(TAPS)


def _sc_call(N, C, H, W):
    CH = C // 2
    CPI = CH + 1
    n_steps = H // RPS

    mesh = plsc.VectorSubcoreMesh(
        core_axis_name="c", subcore_axis_name="s", num_cores=NC,
        num_subcores=NS)

    @functools.partial(
        pl.kernel,
        mesh=mesh,
        compiler_params=pltpu.CompilerParams(
            needs_layout_passes=False, use_tc_tiling_on_sc=False),
        out_type=jax.ShapeDtypeStruct(
            (N, NS, H // RPS, NT, RPS, SW), jnp.float32),
        scratch_types=[
            pltpu.VMEM((SLOTS * WIN, CPI), jnp.int32),
            pltpu.VMEM((2, C, RPS * SW), jnp.float32),
            pltpu.VMEM((2, 2, RPS, SW), jnp.int32),
            pltpu.VMEM((2, NT, RPS, SW), jnp.float32),
        ] + [pltpu.SemaphoreType.DMA] * 8,
    )
    def call(f1_hbm, f2_hbm, ofs_hbm, out_hbm, win, f1b, ofsb, outb,
             rs0, rs1, fs0, fs1, os0, os1, ws0, ws1):
        n = lax.axis_index("c")
        tec = lax.axis_index("s")
        x0 = tec * SW
        xs = jnp.clip(x0 - 2, 0, W - WIN)
        img = n * H * W
        lane = lax.iota(jnp.int32, L)
        zero_i = lane * 0
        one_i = zero_i + 1
        r1 = (lane >= (2 * L - SW)).astype(jnp.int32)
        lane_r = [zero_i, r1, one_i]
        lane_x = [lane, lane + L - SW * r1, lane + 2 * L - SW]
        tap_id = [zero_i + t for t in range(NT)]
        comp0 = zero_i
        comp1 = one_i

        def ring_fire(row, sem):
            r = jnp.minimum(row, H - 1)
            return pltpu.async_copy(
                f2_hbm.at[pl.ds(img + r * W + xs, WIN), :],
                win.at[pl.ds((r & (SLOTS - 1)) * WIN, WIN), :], sem)

        def f1_fire(s, buf, sem):
            y = jnp.minimum(s * RPS, H - RPS)
            for r in range(RPS):
                pltpu.async_copy(
                    f1_hbm.at[n, :, y + r, pl.ds(x0, SW)],
                    f1b.at[buf, :, pl.ds(r * SW, SW)], sem)

        def ofs_fire(s, buf, sem):
            ss = jnp.minimum(s, H // RPS - 1)
            pltpu.async_copy(ofs_hbm.at[n, tec, ss], ofsb.at[buf], sem)

        def out_fire(s, buf, sem):
            return pltpu.async_copy(
                outb.at[buf], out_hbm.at[n, tec, s], sem)

        def ring_wait(sem):
            pltpu.make_async_copy(
                f2_hbm.at[pl.ds(0, WIN), :],
                win.at[pl.ds(0, WIN), :], sem).wait()

        def f1_wait(sem):
            for r in range(RPS):
                pltpu.make_async_copy(
                    f1_hbm.at[0, :, 0, pl.ds(0, SW)],
                    f1b.at[0, :, pl.ds(0, SW)], sem).wait()

        def ofs_wait(sem):
            pltpu.make_async_copy(
                ofs_hbm.at[0, 0, 0], ofsb.at[0], sem).wait()

        def out_wait(sem):
            pltpu.make_async_copy(
                outb.at[0], out_hbm.at[0, 0, 0], sem).wait()

        for k in range(SLOTS - 4):
            pltpu.sync_copy(
                f2_hbm.at[pl.ds(img + k * W + xs, WIN), :],
                win.at[pl.ds(k * WIN, WIN), :])
        for r in range(RPS):
            pltpu.sync_copy(
                f1_hbm.at[n, :, r, pl.ds(x0, SW)],
                f1b.at[0, :, pl.ds(r * SW, SW)])
        pltpu.sync_copy(ofs_hbm.at[n, tec, 0], ofsb.at[0])

        def compute(y, p):
            for g in range(GRP):
                ofx = plsc.load_gather(
                    ofsb.at[p], [comp0, lane_r[g], lane_x[g]])
                ofy = plsc.load_gather(
                    ofsb.at[p], [comp1, lane_r[g], lane_x[g]])
                gx0 = x0 + lane_x[g] + ofx
                gy0 = y + lane_r[g] + ofy
                winrow = []
                for (dy, dx) in TAPS:
                    sxl = jnp.clip((gx0 + dx) - xs, 0, WIN - 1)
                    slot = (gy0 + dy) & (SLOTS - 1)
                    winrow.append(slot * WIN + sxl)
                xg = g * L

                def c_body(c2, accs):
                    cs = comp0 + c2
                    f1e = f1b[p, 2 * c2, pl.ds(xg, L)]
                    f1o = f1b[p, 2 * c2 + 1, pl.ds(xg, L)]
                    new = []
                    for t in range(NT):
                        w = plsc.load_gather(win, [winrow[t], cs])
                        lo, hi = plsc.unpack(
                            plsc.bitcast(w, jnp.bfloat16),
                            format=plsc.PackFormat.INTERLEAVED)
                        new.append(accs[t] + (f1e * lo + f1o * hi))
                    return tuple(new)

                accs = lax.fori_loop(
                    0, CH, c_body,
                    tuple(jnp.zeros((L,), jnp.float32) for _ in range(NT)))
                zero = jnp.zeros((L,), jnp.float32)
                for t, (dy, dx) in enumerate(TAPS):
                    gx = plsc.bitcast(gx0 + dx, jnp.uint32)
                    gy = plsc.bitcast(gy0 + dy, jnp.uint32)
                    m = (gx < W) & (gy < H)
                    plsc.store_scatter(
                        outb.at[p], [tap_id[t], lane_r[g], lane_x[g]],
                        jnp.where(m, accs[t], zero))

        def step(t, _):
            for p in range(2):
                s = 2 * t + p
                y = RPS * s
                not_first = t >= 1

                if p == 0:
                    @pl.when(not_first)
                    def _():
                        f1_wait(fs0)
                        ofs_wait(os0)
                else:
                    f1_wait(fs1)
                    ofs_wait(os1)

                @pl.when(not_first)
                def _():
                    ring_wait(rs0 if p == 0 else rs1)
                    ring_wait(rs0 if p == 0 else rs1)
                    out_wait(ws0 if p == 0 else ws1)

                ring_fire(y + 12, rs0 if p == 0 else rs1)
                ring_fire(y + 13, rs0 if p == 0 else rs1)
                f1_fire(s + 1, 1 - p, fs1 if p == 0 else fs0)
                ofs_fire(s + 1, 1 - p, os1 if p == 0 else os0)

                compute(y, p)
                out_fire(s, p, ws0 if p == 0 else ws1)
            return 0

        lax.fori_loop(0, n_steps // 2, step, 0)

        ring_wait(rs0)
        ring_wait(rs0)
        ring_wait(rs1)
        ring_wait(rs1)
        f1_wait(fs0)
        ofs_wait(os0)
        out_wait(ws0)
        out_wait(ws1)

    return call


def kernel(f1, f2, ofs):
    N, C, H, W = f1.shape
    H2 = H // RPS
    assert N == NC and W == NS * SW and H % (2 * RPS) == 0 and C % 2 == 0
    f2p = lax.bitcast_convert_type(
        jnp.transpose(f2, (0, 2, 3, 1)).astype(jnp.bfloat16).reshape(
            N * H * W, C // 2, 2),
        jnp.int32)
    f2p = jnp.concatenate(
        [f2p, jnp.zeros((N * H * W, 1), jnp.int32)], axis=1)
    ofs_q = ofs.reshape(N, 2, H2, RPS, NS, SW).transpose(0, 4, 2, 1, 3, 5)
    out_t = _sc_call(N, C, H, W)(f1, f2p, ofs_q)
    return out_t.transpose(0, 3, 2, 4, 1, 5).reshape(N, NT, H, W)


# speedup vs baseline: 1.2629x; 1.1676x over previous
import functools

import jax
import jax.numpy as jnp
from jax import lax
from jax.experimental import pallas as pl
from jax.experimental.pallas import tpu as pltpu
from jax.experimental.pallas import tpu_sc as plsc

RX = 4
RY = 4
NC = 2
NS = 16
L = 16
SLOTS = 16
SW = 24
WIN = SW + 10
RPS = 2
GRP = RPS * SW // L
TAPS = [(j - RY // 2, i - RX // 2) for j in range(RY) for i in range(RX)]
NT = len(TAPS)


def _sc_call(N, C, H, W):
    CH = C // 2
    CPI = CH + 1
    n_steps = H // RPS

    mesh = plsc.VectorSubcoreMesh(
        core_axis_name="c", subcore_axis_name="s", num_cores=NC,
        num_subcores=NS)

    @functools.partial(
        pl.kernel,
        mesh=mesh,
        compiler_params=pltpu.CompilerParams(
            needs_layout_passes=False, use_tc_tiling_on_sc=False),
        out_type=jax.ShapeDtypeStruct((N, NT, H, W), jnp.float32),
        scratch_types=[
            pltpu.VMEM((SLOTS * WIN, CPI), jnp.int32),
            pltpu.VMEM((2, C, RPS * SW), jnp.float32),
            pltpu.VMEM((2, 2, RPS, SW), jnp.int32),
            pltpu.VMEM((2, NT, RPS, SW), jnp.float32),
        ] + [pltpu.SemaphoreType.DMA] * 8,
    )
    def call(f1_hbm, f2_hbm, ofs_hbm, out_hbm, win, f1b, ofsb, outb,
             rs0, rs1, fs0, fs1, os0, os1, ws0, ws1):
        n = lax.axis_index("c")
        tec = lax.axis_index("s")
        x0 = tec * SW
        xs = jnp.clip(x0 - 2, 0, W - WIN)
        img = n * H * W
        lane = lax.iota(jnp.int32, L)
        zero_i = lane * 0
        one_i = zero_i + 1
        r1 = (lane >= (2 * L - SW)).astype(jnp.int32)
        lane_r = [zero_i, r1, one_i]
        lane_x = [lane, lane + L - SW * r1, lane + 2 * L - SW]
        tap_id = [zero_i + t for t in range(NT)]
        comp0 = zero_i
        comp1 = one_i

        def ring_fire(row, sem):
            r = jnp.minimum(row, H - 1)
            return pltpu.async_copy(
                f2_hbm.at[pl.ds(img + r * W + xs, WIN), :],
                win.at[pl.ds((r & (SLOTS - 1)) * WIN, WIN), pl.ds(0, CH)],
                sem)

        def f1_fire(s, buf, sem):
            y = jnp.minimum(s * RPS, H - RPS)
            for r in range(RPS):
                pltpu.async_copy(
                    f1_hbm.at[n, :, y + r, pl.ds(x0, SW)],
                    f1b.at[buf, :, pl.ds(r * SW, SW)], sem)

        def ofs_fire(s, buf, sem):
            yy = jnp.minimum(s * RPS, H - RPS)
            pltpu.async_copy(
                ofs_hbm.at[n, :, pl.ds(yy, RPS), pl.ds(x0, SW)],
                ofsb.at[buf], sem)

        def out_fire(s, buf, sem):
            return pltpu.async_copy(
                outb.at[buf],
                out_hbm.at[n, :, pl.ds(s * RPS, RPS), pl.ds(x0, SW)], sem)

        def ring_wait(sem):
            pltpu.make_async_copy(
                f2_hbm.at[pl.ds(0, WIN), :],
                win.at[pl.ds(0, WIN), pl.ds(0, CH)], sem).wait()

        def f1_wait(sem):
            for r in range(RPS):
                pltpu.make_async_copy(
                    f1_hbm.at[0, :, 0, pl.ds(0, SW)],
                    f1b.at[0, :, pl.ds(0, SW)], sem).wait()

        def ofs_wait(sem):
            pltpu.make_async_copy(
                ofs_hbm.at[0, :, pl.ds(0, RPS), pl.ds(0, SW)],
                ofsb.at[0], sem).wait()

        def out_wait(sem):
            pltpu.make_async_copy(
                outb.at[0],
                out_hbm.at[0, :, pl.ds(0, RPS), pl.ds(0, SW)], sem).wait()

        for k in range(SLOTS - 4):
            pltpu.sync_copy(
                f2_hbm.at[pl.ds(img + k * W + xs, WIN), :],
                win.at[pl.ds(k * WIN, WIN), pl.ds(0, CH)])
        for r in range(RPS):
            pltpu.sync_copy(
                f1_hbm.at[n, :, r, pl.ds(x0, SW)],
                f1b.at[0, :, pl.ds(r * SW, SW)])
        pltpu.sync_copy(
            ofs_hbm.at[n, :, pl.ds(0, RPS), pl.ds(x0, SW)], ofsb.at[0])

        def compute(y, p):
            for g in range(GRP):
                ofx = plsc.load_gather(
                    ofsb.at[p], [comp0, lane_r[g], lane_x[g]])
                ofy = plsc.load_gather(
                    ofsb.at[p], [comp1, lane_r[g], lane_x[g]])
                gx0 = x0 + lane_x[g] + ofx
                gy0 = y + lane_r[g] + ofy
                winrow = []
                for (dy, dx) in TAPS:
                    sxl = jnp.clip((gx0 + dx) - xs, 0, WIN - 1)
                    slot = (gy0 + dy) & (SLOTS - 1)
                    winrow.append(slot * WIN + sxl)
                xg = g * L

                def c_body(c2, accs):
                    cs = comp0 + c2
                    f1e = f1b[p, 2 * c2, pl.ds(xg, L)]
                    f1o = f1b[p, 2 * c2 + 1, pl.ds(xg, L)]
                    new = []
                    for t in range(NT):
                        w = plsc.load_gather(win, [winrow[t], cs])
                        lo, hi = plsc.unpack(
                            plsc.bitcast(w, jnp.bfloat16),
                            format=plsc.PackFormat.INTERLEAVED)
                        new.append(accs[t] + (f1e * lo + f1o * hi))
                    return tuple(new)

                accs = lax.fori_loop(
                    0, CH, c_body,
                    tuple(jnp.zeros((L,), jnp.float32) for _ in range(NT)))
                zero = jnp.zeros((L,), jnp.float32)
                for t, (dy, dx) in enumerate(TAPS):
                    gx = plsc.bitcast(gx0 + dx, jnp.uint32)
                    gy = plsc.bitcast(gy0 + dy, jnp.uint32)
                    m = (gx < W) & (gy < H)
                    plsc.store_scatter(
                        outb.at[p], [tap_id[t], lane_r[g], lane_x[g]],
                        jnp.where(m, accs[t], zero))

        def step(t, _):
            for p in range(2):
                s = 2 * t + p
                y = RPS * s
                not_first = t >= 1

                if p == 0:
                    @pl.when(not_first)
                    def _():
                        f1_wait(fs0)
                        ofs_wait(os0)
                else:
                    f1_wait(fs1)
                    ofs_wait(os1)

                @pl.when(not_first)
                def _():
                    ring_wait(rs0 if p == 0 else rs1)
                    ring_wait(rs0 if p == 0 else rs1)
                    out_wait(ws0 if p == 0 else ws1)

                ring_fire(y + 12, rs0 if p == 0 else rs1)
                ring_fire(y + 13, rs0 if p == 0 else rs1)
                f1_fire(s + 1, 1 - p, fs1 if p == 0 else fs0)
                ofs_fire(s + 1, 1 - p, os1 if p == 0 else os0)

                compute(y, p)
                out_fire(s, p, ws0 if p == 0 else ws1)
            return 0

        lax.fori_loop(0, n_steps // 2, step, 0)

        ring_wait(rs0)
        ring_wait(rs0)
        ring_wait(rs1)
        ring_wait(rs1)
        f1_wait(fs0)
        ofs_wait(os0)
        out_wait(ws0)
        out_wait(ws1)

    return call


def kernel(f1, f2, ofs):
    N, C, H, W = f1.shape
    assert N == NC and W == NS * SW and H % (2 * RPS) == 0 and C % 2 == 0
    f2p = lax.bitcast_convert_type(
        jnp.transpose(f2, (0, 2, 3, 1)).astype(jnp.bfloat16).reshape(
            N * H * W, C // 2, 2),
        jnp.int32)
    return _sc_call(N, C, H, W)(f1, f2p, ofs)


# speedup vs baseline: 1.2636x; 1.0006x over previous
import functools

import jax
import jax.numpy as jnp
from jax import lax
from jax.experimental import pallas as pl
from jax.experimental.pallas import tpu as pltpu
from jax.experimental.pallas import tpu_sc as plsc

RX = 4
RY = 4
NC = 2
NS = 16
L = 16
SLOTS = 16
SW = 24
WIN = SW + 10
RPS = 2
GRP = RPS * SW // L
TAPS = [(j - RY // 2, i - RX // 2) for j in range(RY) for i in range(RX)]
NT = len(TAPS)


def _sc_call(N, C, H, W):
    CH = C // 2
    CPI = CH + 1
    n_steps = H // RPS

    mesh = plsc.VectorSubcoreMesh(
        core_axis_name="c", subcore_axis_name="s", num_cores=NC,
        num_subcores=NS)

    @functools.partial(
        pl.kernel,
        mesh=mesh,
        compiler_params=pltpu.CompilerParams(
            needs_layout_passes=False, use_tc_tiling_on_sc=False),
        out_type=jax.ShapeDtypeStruct((N, NT, H, W), jnp.float32),
        scratch_types=[
            pltpu.VMEM((SLOTS * WIN, CPI), jnp.int32),
            pltpu.VMEM((2, C, RPS * SW), jnp.float32),
            pltpu.VMEM((2, 2, RPS, SW), jnp.int32),
            pltpu.VMEM((2, NT, RPS, SW), jnp.float32),
        ] + [pltpu.SemaphoreType.DMA] * 8,
    )
    def call(f1_hbm, f2_hbm, ofs_hbm, out_hbm, win, f1b, ofsb, outb,
             rs0, rs1, fs0, fs1, os0, os1, ws0, ws1):
        n = lax.axis_index("c")
        tec = lax.axis_index("s")
        x0 = tec * SW
        xs = jnp.clip(x0 - 2, 0, W - WIN)
        img = n * H * W
        lane = lax.iota(jnp.int32, L)
        zero_i = lane * 0
        one_i = zero_i + 1
        r1 = (lane >= (2 * L - SW)).astype(jnp.int32)
        lane_r = [zero_i, r1, one_i]
        lane_x = [lane, lane + L - SW * r1, lane + 2 * L - SW]
        tap_id = [zero_i + t for t in range(NT)]
        comp0 = zero_i
        comp1 = one_i

        def ring_fire(row, sem):
            r = jnp.minimum(row, H - 1)
            return pltpu.async_copy(
                f2_hbm.at[pl.ds(img + r * W + xs, WIN), :],
                win.at[pl.ds((r & (SLOTS - 1)) * WIN, WIN), pl.ds(0, CH)],
                sem)

        def f1_fire(s, buf, sem):
            y = jnp.minimum(s * RPS, H - RPS)
            for r in range(RPS):
                pltpu.async_copy(
                    f1_hbm.at[n, :, y + r, pl.ds(x0, SW)],
                    f1b.at[buf, :, pl.ds(r * SW, SW)], sem)

        def ofs_fire(s, buf, sem):
            yy = jnp.minimum(s * RPS, H - RPS)
            pltpu.async_copy(
                ofs_hbm.at[n, :, pl.ds(yy, RPS), pl.ds(x0, SW)],
                ofsb.at[buf], sem)

        def out_fire(s, buf, sem):
            return pltpu.async_copy(
                outb.at[buf],
                out_hbm.at[n, :, pl.ds(s * RPS, RPS), pl.ds(x0, SW)], sem)

        def ring_wait(sem):
            pltpu.make_async_copy(
                f2_hbm.at[pl.ds(0, WIN), :],
                win.at[pl.ds(0, WIN), pl.ds(0, CH)], sem).wait()

        def f1_wait(sem):
            for r in range(RPS):
                pltpu.make_async_copy(
                    f1_hbm.at[0, :, 0, pl.ds(0, SW)],
                    f1b.at[0, :, pl.ds(0, SW)], sem).wait()

        def ofs_wait(sem):
            pltpu.make_async_copy(
                ofs_hbm.at[0, :, pl.ds(0, RPS), pl.ds(0, SW)],
                ofsb.at[0], sem).wait()

        def out_wait(sem):
            pltpu.make_async_copy(
                outb.at[0],
                out_hbm.at[0, :, pl.ds(0, RPS), pl.ds(0, SW)], sem).wait()

        for k in range(SLOTS - 4):
            pltpu.sync_copy(
                f2_hbm.at[pl.ds(img + k * W + xs, WIN), :],
                win.at[pl.ds(k * WIN, WIN), pl.ds(0, CH)])
        for r in range(RPS):
            pltpu.sync_copy(
                f1_hbm.at[n, :, r, pl.ds(x0, SW)],
                f1b.at[0, :, pl.ds(r * SW, SW)])
        pltpu.sync_copy(
            ofs_hbm.at[n, :, pl.ds(0, RPS), pl.ds(x0, SW)], ofsb.at[0])

        def compute(y, p):
            for g in range(GRP):
                ofx = plsc.load_gather(
                    ofsb.at[p], [comp0, lane_r[g], lane_x[g]])
                ofy = plsc.load_gather(
                    ofsb.at[p], [comp1, lane_r[g], lane_x[g]])
                gx0 = x0 + lane_x[g] + ofx
                gy0 = y + lane_r[g] + ofy
                winrow = []
                for (dy, dx) in TAPS:
                    sxl = jnp.clip((gx0 + dx) - xs, 0, WIN - 1)
                    slot = (gy0 + dy) & (SLOTS - 1)
                    winrow.append(slot * WIN + sxl)
                xg = g * L

                def c_body(c2, accs):
                    cs = comp0 + c2
                    f1e = f1b[p, 2 * c2, pl.ds(xg, L)]
                    f1o = f1b[p, 2 * c2 + 1, pl.ds(xg, L)]
                    new = []
                    for t in range(NT):
                        w = plsc.load_gather(win, [winrow[t], cs])
                        lo, hi = plsc.unpack(
                            plsc.bitcast(w, jnp.bfloat16),
                            format=plsc.PackFormat.INTERLEAVED)
                        new.append(accs[t] + (f1e * lo + f1o * hi))
                    return tuple(new)

                accs = lax.fori_loop(
                    0, CH, c_body,
                    tuple(jnp.zeros((L,), jnp.float32) for _ in range(NT)))
                zero = jnp.zeros((L,), jnp.float32)
                for t, (dy, dx) in enumerate(TAPS):
                    gx = plsc.bitcast(gx0 + dx, jnp.uint32)
                    gy = plsc.bitcast(gy0 + dy, jnp.uint32)
                    m = (gx < W) & (gy < H)
                    plsc.store_scatter(
                        outb.at[p], [tap_id[t], lane_r[g], lane_x[g]],
                        jnp.where(m, accs[t], zero))

        def step(t, _):
            for p in range(2):
                s = 2 * t + p
                y = RPS * s
                not_first = t >= 1

                if p == 0:
                    @pl.when(not_first)
                    def _():
                        f1_wait(fs0)
                        ofs_wait(os0)
                else:
                    f1_wait(fs1)
                    ofs_wait(os1)

                @pl.when(not_first)
                def _():
                    ring_wait(rs0 if p == 0 else rs1)
                    ring_wait(rs0 if p == 0 else rs1)
                    out_wait(ws0 if p == 0 else ws1)

                ring_fire(y + 12, rs0 if p == 0 else rs1)
                ring_fire(y + 13, rs0 if p == 0 else rs1)
                f1_fire(s + 1, 1 - p, fs1 if p == 0 else fs0)
                ofs_fire(s + 1, 1 - p, os1 if p == 0 else os0)

                compute(y, p)
                out_fire(s, p, ws0 if p == 0 else ws1)
            return 0

        lax.fori_loop(0, n_steps // 2, step, 0)

        ring_wait(rs0)
        ring_wait(rs0)
        ring_wait(rs1)
        ring_wait(rs1)
        f1_wait(fs0)
        ofs_wait(os0)
        out_wait(ws0)
        out_wait(ws1)

    return call


def kernel(f1, f2, ofs):
    N, C, H, W = f1.shape
    assert N == NC and W == NS * SW and H % (2 * RPS) == 0 and C % 2 == 0
    f2p = lax.bitcast_convert_type(
        jnp.transpose(f2.astype(jnp.bfloat16), (0, 2, 3, 1)).reshape(
            N * H * W, C // 2, 2),
        jnp.int32)
    return _sc_call(N, C, H, W)(f1, f2p, ofs)
